# baseline (speedup 1.0000x reference)
import numpy as np
import jax
import jax.numpy as jnp
from jax.experimental import pallas as pl

_BATCH = 16384
_D = 256
_NF = 33
_B = 256
_N = 1024


_SRC64 = [0, 1, 2, 3, 4, 5, 6, 11]
_SRC16 = ([7, 8] + [17 + j for j in range(7)] + [24 + j for j in range(9)]
          + [9, 10, 12, 13, 14, 15, 16])
_SRCC = [71, 72] + [81 + j for j in range(7)]
_CSCALE = [1.0 / 100, 1.0 / 1023] + [0.5] * 7


def _static_maps():
    src = np.full(_N, -1, np.int64)
    colv = np.full(_N, -1.0, np.float32)
    scalev = np.zeros(_N, np.float32)
    for i, s in enumerate(_SRC64):
        src[64 * i:64 * (i + 1)] = s
        colv[64 * i:64 * (i + 1)] = np.arange(64)
    for j, s in enumerate(_SRC16):
        b = 512 + 16 * j
        src[b:b + 16] = s
        colv[b:b + 16] = np.arange(16)
    for k, s in enumerate(_SRCC):
        src[912 + k] = s
        scalev[912 + k] = _CSCALE[k]
    S = np.zeros((128, _N), np.float32)
    valid = src >= 0
    S[src[valid], np.where(valid)[0]] = 1.0
    andm = np.full(128, 63, np.int32)
    andm[24:33] = 15
    clampm = np.full(128, 63, np.int32)
    for c, lim in [(9, 4), (10, 8), (12, 2), (13, 8), (14, 4), (15, 2),
                   (16, 2)]:
        clampm[c] = lim
    clampm[17:24] = 7
    sqrtm = np.zeros(128, np.int32)
    sqrtm[7] = sqrtm[8] = 1
    return S, colv, scalev, andm, clampm, sqrtm


_S_NP, _COLV_NP, _SCALEV_NP, _ANDM_NP, _CLAMPM_NP, _SQRTM_NP = _static_maps()


def _code_matrix():
    code = np.zeros((_N, 734), np.float32)
    def oh(m, n):
        z = np.zeros(n, np.float32)
        if 0 <= m < n:
            z[m] = 1.0
        return z
    for v in range(64):
        code[v, 0:512] = 0.0
        code[448 + v, 609:625] = oh(v, 16)
    for s in range(16):
        code[512 + s, 512:523] = oh(min(s, 10), 11)
        code[528 + s, 523:555] = oh(min(s, 31), 32)
    for j in range(7):
        for m in range(16):
            code[544 + 16 * j + m, 643 + 13 * j:643 + 13 * (j + 1)] = \
                oh(m + 6, 13)
    for j in range(9):
        nb = min(4, 33 - 4 * j)
        for m in range(16):
            for b in range(nb):
                code[656 + 16 * j + m, 555 + 4 * j + b] = float((m >> b) & 1)
    for m in range(16):
        code[800 + m, 597:601] = oh(m, 4)
        code[816 + m, 601:609] = oh(m, 8)
        code[832 + m, 625:627] = oh(m, 2)
        code[848 + m, 627:635] = oh(m, 8)
        code[864 + m, 635:639] = oh(m, 4)
        code[880 + m, 639:641] = oh(m, 2)
        code[896 + m, 641:643] = oh(m, 2)
    code[912, 588] = 1.0
    code[913, 589] = 1.0
    for j in range(7):
        code[914 + j, 590 + j] = 1.0
    return code


_CODE_NP = _code_matrix()


def _build_lut(species_emb, abilities_emb, items_emb, actions_emb,
               ability_onehot, item_onehot, species_onehot,
               W_ab, W_it, W_enc):
    L = jnp.asarray(_CODE_NP) @ W_enc
    L = L.at[0:64].add(species_emb[:64] + species_onehot[:64] @ W_enc[:512])
    L = L.at[64:128].add(abilities_emb[:64] + items_emb[:64]
                         + ability_onehot[:64] @ W_ab)
    L = L.at[128:192].add(item_onehot[:64] @ W_it)
    for k in range(3, 7):
        L = L.at[64 * k:64 * (k + 1)].add(actions_emb[:64])
    return L


def _encoder_block(e_ref, s_ref, colv_ref, scalev_ref, andm_ref, clampm_ref,
                   sqrtm_ref, lhi_ref, bias_ref, scale_ref,
                   lnb_ref, o_ref):
    raw = e_ref[...]
    t = jnp.minimum(raw & andm_ref[...], clampm_ref[...])
    sq = ((raw >= 1).astype(jnp.int32) + (raw >= 4).astype(jnp.int32)
          + (raw >= 9).astype(jnp.int32) + (raw >= 16).astype(jnp.int32)
          + (raw >= 25).astype(jnp.int32) + (raw >= 36).astype(jnp.int32)
          + (raw >= 49).astype(jnp.int32))
    t = jnp.where(sqrtm_ref[...] == 1, sq, t)
    E = jnp.dot(t.astype(jnp.bfloat16), s_ref[...],
                preferred_element_type=jnp.float32)
    oh = (E == colv_ref[...]).astype(jnp.float32)
    X = oh + E * scalev_ref[...]
    acc = jnp.broadcast_to(bias_ref[...], (_B, _D))
    acc = acc + jnp.dot(X, lhi_ref[...], preferred_element_type=jnp.float32)
    mu = jnp.mean(acc, axis=1, keepdims=True)
    d = acc - mu
    var = jnp.mean(d * d, axis=1, keepdims=True)
    o_ref[...] = d * jax.lax.rsqrt(var + 1e-6) * scale_ref[...] + lnb_ref[...]


def kernel(entity, species_emb, abilities_emb, items_emb, actions_emb,
           ability_onehot, item_onehot, species_onehot, W_ab, b_ab,
           W_it, b_it, W_enc, b_enc, ln_scale, ln_bias):
    L = jnp.zeros((_N, _D), jnp.float32)
    bias = (b_ab + b_it + b_enc).reshape(1, _D)
    scale = ln_scale.reshape(1, _D)
    lnb = ln_bias.reshape(1, _D)
    e_pad = (jnp.zeros((_BATCH, 128), jnp.int32)
             .at[:, :_NF].set(entity)
             .at[:, 64:64 + _NF].set(entity))
    S = jnp.asarray(_S_NP, jnp.bfloat16)
    colv = jnp.asarray(_COLV_NP).reshape(1, _N)
    scalev = jnp.asarray(_SCALEV_NP).reshape(1, _N)
    andm = jnp.asarray(_ANDM_NP).reshape(1, 128)
    clampm = jnp.asarray(_CLAMPM_NP).reshape(1, 128)
    sqrtm = jnp.asarray(_SQRTM_NP).reshape(1, 128)
    const = lambda i: (0, 0)
    return pl.pallas_call(
        _encoder_block,
        grid=(_BATCH // _B,),
        in_specs=[
            pl.BlockSpec((_B, 128), lambda i: (i, 0)),
            pl.BlockSpec((128, _N), const),
            pl.BlockSpec((1, _N), const),
            pl.BlockSpec((1, _N), const),
            pl.BlockSpec((1, 128), const),
            pl.BlockSpec((1, 128), const),
            pl.BlockSpec((1, 128), const),
            pl.BlockSpec((_N, _D), const),
            pl.BlockSpec((1, _D), const),
            pl.BlockSpec((1, _D), const),
            pl.BlockSpec((1, _D), const),
        ],
        out_specs=pl.BlockSpec((_B, _D), lambda i: (i, 0)),
        out_shape=jax.ShapeDtypeStruct((_BATCH, _D), jnp.float32),
    )(e_pad, S, colv, scalev, andm, clampm, sqrtm, L,
      bias, scale, lnb)


# speedup vs baseline: 1.0311x; 1.0311x over previous
import numpy as np
import jax
import jax.numpy as jnp
from jax.experimental import pallas as pl

_BATCH = 16384
_D = 256
_NF = 33
_B = 256
_N = 1024

_SRC64 = [0, 1, 2, 3, 4, 5, 6, 11]
_SRC16 = ([7, 8] + [17 + j for j in range(7)] + [24 + j for j in range(9)]
          + [9, 10, 12, 13, 14, 15, 16])


def _static_maps():
    src = np.full(_N, -1, np.int64)
    colv = np.full(_N, -1.0, np.float32)
    for i, s in enumerate(_SRC64):
        src[64 * i:64 * (i + 1)] = s
        colv[64 * i:64 * (i + 1)] = np.arange(64)
    for j, s in enumerate(_SRC16):
        b = 512 + 16 * j
        src[b:b + 16] = s
        colv[b:b + 16] = np.arange(16)
    S = np.zeros((_NF, _N), np.float32)
    valid = src >= 0
    S[src[valid], np.where(valid)[0]] = 1.0
    andm = np.full(_NF, 63, np.int32)
    andm[24:33] = 15
    clampm = np.full(_NF, 63, np.int32)
    for c, lim in [(9, 4), (10, 8), (12, 2), (13, 8), (14, 4), (15, 2),
                   (16, 2)]:
        clampm[c] = lim
    clampm[17:24] = 7
    sqrtm = np.zeros(_NF, np.int32)
    sqrtm[7] = sqrtm[8] = 1
    return S, colv, andm, clampm, sqrtm


_S_NP, _COLV_NP, _ANDM_NP, _CLAMPM_NP, _SQRTM_NP = _static_maps()

_BITS_NP = np.array([[(m >> b) & 1 for b in range(4)] for m in range(16)],
                    np.float32)


def _build_luts(species_emb, abilities_emb, items_emb, actions_emb,
                ability_onehot, item_onehot, species_onehot,
                W_ab, W_it, W_enc):
    z = jnp.zeros((16, _D), jnp.float32)
    bits = jnp.asarray(_BITS_NP)
    blocks = [
        species_emb[:64] + species_onehot[:64] @ W_enc[:512],
        (abilities_emb[:64] + items_emb[:64]
         + ability_onehot[:64] @ W_ab),
        item_onehot[:64] @ W_it,
        actions_emb[:64], actions_emb[:64],
        actions_emb[:64], actions_emb[:64],
        jnp.concatenate([W_enc[609:625], jnp.zeros((48, _D))]),
        jnp.concatenate([W_enc[512:523],
                         jnp.broadcast_to(W_enc[522:523], (5, _D))]),
        W_enc[523:539],
    ]
    for j in range(7):
        blocks.append(jnp.concatenate(
            [W_enc[649 + 13 * j:656 + 13 * j], jnp.zeros((9, _D))]))
    for j in range(9):
        nb = min(4, 33 - 4 * j)
        blocks.append(bits[:, :nb] @ W_enc[555 + 4 * j:555 + 4 * j + nb])
    for off, w in [(597, 4), (601, 8), (625, 2), (627, 8), (635, 4),
                   (639, 2), (641, 2)]:
        blocks.append(jnp.concatenate(
            [W_enc[off:off + w], jnp.zeros((16 - w, _D))]))
    blocks.append(jnp.zeros((_N - 912, _D)))
    L = jnp.concatenate(blocks)
    Lc = jnp.zeros((_NF, _D), jnp.float32)
    Lc = Lc.at[7].set(W_enc[588] / 100.0)
    Lc = Lc.at[8].set(W_enc[589] / 1023.0)
    Lc = Lc.at[17:24].set(0.5 * W_enc[590:597])
    return L, Lc


def _encoder_block(e_ref, s_ref, colv_ref, andm_ref, clampm_ref,
                   sqrtm_ref, l_ref, lc_ref, bias_ref, scale_ref,
                   lnb_ref, o_ref):
    raw = e_ref[...]
    t = jnp.minimum(raw & andm_ref[...], clampm_ref[...])
    sq = ((raw >= 1).astype(jnp.int32) + (raw >= 4).astype(jnp.int32)
          + (raw >= 9).astype(jnp.int32) + (raw >= 16).astype(jnp.int32)
          + (raw >= 25).astype(jnp.int32) + (raw >= 36).astype(jnp.int32)
          + (raw >= 49).astype(jnp.int32))
    t = jnp.where(sqrtm_ref[...] == 1, sq, t)
    E = jnp.dot(t.astype(jnp.bfloat16), s_ref[...],
                preferred_element_type=jnp.float32)
    oh = (E == colv_ref[...]).astype(jnp.float32)
    acc = jnp.broadcast_to(bias_ref[...], (_B, _D))
    acc = acc + jnp.dot(raw.astype(jnp.float32), lc_ref[...],
                        preferred_element_type=jnp.float32)
    acc = acc + jnp.dot(oh, l_ref[...], preferred_element_type=jnp.float32)
    mu = jnp.mean(acc, axis=1, keepdims=True)
    d = acc - mu
    var = jnp.mean(d * d, axis=1, keepdims=True)
    o_ref[...] = d * jax.lax.rsqrt(var + 1e-6) * scale_ref[...] + lnb_ref[...]


def kernel(entity, species_emb, abilities_emb, items_emb, actions_emb,
           ability_onehot, item_onehot, species_onehot, W_ab, b_ab,
           W_it, b_it, W_enc, b_enc, ln_scale, ln_bias):
    L, Lc = _build_luts(species_emb, abilities_emb, items_emb, actions_emb,
                        ability_onehot, item_onehot, species_onehot,
                        W_ab, W_it, W_enc)
    bias = (b_ab + b_it + b_enc).reshape(1, _D)
    scale = ln_scale.reshape(1, _D)
    lnb = ln_bias.reshape(1, _D)
    S = jnp.asarray(_S_NP, jnp.bfloat16)
    colv = jnp.asarray(_COLV_NP).reshape(1, _N)
    andm = jnp.asarray(_ANDM_NP).reshape(1, _NF)
    clampm = jnp.asarray(_CLAMPM_NP).reshape(1, _NF)
    sqrtm = jnp.asarray(_SQRTM_NP).reshape(1, _NF)
    const = lambda i: (0, 0)
    return pl.pallas_call(
        _encoder_block,
        grid=(_BATCH // _B,),
        in_specs=[
            pl.BlockSpec((_B, _NF), lambda i: (i, 0)),
            pl.BlockSpec((_NF, _N), const),
            pl.BlockSpec((1, _N), const),
            pl.BlockSpec((1, _NF), const),
            pl.BlockSpec((1, _NF), const),
            pl.BlockSpec((1, _NF), const),
            pl.BlockSpec((_N, _D), const),
            pl.BlockSpec((_NF, _D), const),
            pl.BlockSpec((1, _D), const),
            pl.BlockSpec((1, _D), const),
            pl.BlockSpec((1, _D), const),
        ],
        out_specs=pl.BlockSpec((_B, _D), lambda i: (i, 0)),
        out_shape=jax.ShapeDtypeStruct((_BATCH, _D), jnp.float32),
    )(entity, S, colv, andm, clampm, sqrtm, L, Lc, bias, scale, lnb)


# speedup vs baseline: 1.7582x; 1.7052x over previous
import numpy as np
import jax
import jax.numpy as jnp
from jax.experimental import pallas as pl

_BATCH = 16384
_D = 256
_NF = 33
_B = 256
_N = 1024

_SRC64 = [0, 1, 2, 3, 4, 5, 6, 11]
_SRC16 = ([7, 8] + [17 + j for j in range(7)] + [24 + j for j in range(9)]
          + [9, 10, 12, 13, 14, 15, 16])


def _static_maps():
    src = np.full(_N, -1, np.int64)
    colv = np.full(_N, -1.0, np.float32)
    for i, s in enumerate(_SRC64):
        src[64 * i:64 * (i + 1)] = s
        colv[64 * i:64 * (i + 1)] = np.arange(64)
    for j, s in enumerate(_SRC16):
        b = 512 + 16 * j
        src[b:b + 16] = s
        colv[b:b + 16] = np.arange(16)
    S = np.zeros((_NF, _N), np.float32)
    valid = src >= 0
    S[src[valid], np.where(valid)[0]] = 1.0
    andm = np.full(_NF, 63, np.int32)
    andm[24:33] = 15
    clampm = np.full(_NF, 63, np.int32)
    for c, lim in [(9, 4), (10, 8), (12, 2), (13, 8), (14, 4), (15, 2),
                   (16, 2)]:
        clampm[c] = lim
    clampm[17:24] = 7
    sqrtm = np.zeros(_NF, np.int32)
    sqrtm[7] = sqrtm[8] = 1
    return S, colv, andm, clampm, sqrtm


_S_NP, _COLV_NP, _ANDM_NP, _CLAMPM_NP, _SQRTM_NP = _static_maps()

def _mix_matrix():
    WE, SP, AB, IT, ACT, WAB, WIT = 0, 734, 798, 862, 926, 990, 1054
    M = np.zeros((_N + _NF + 7, 1118), np.float32)
    for v in range(64):
        M[v, SP + v] = 1.0
        M[v, WE + v] = 1.0
        M[64 + v, AB + v] = 1.0
        M[64 + v, IT + v] = 1.0
        M[64 + v, WAB + v] = 1.0
        M[128 + v, WIT + v] = 1.0
        for k in range(4):
            M[192 + 64 * k + v, ACT + v] = 1.0
        if v < 16:
            M[448 + v, WE + 609 + v] = 1.0
    for s in range(16):
        M[512 + s, WE + 512 + min(s, 10)] = 1.0
        M[528 + s, WE + 523 + min(s, 31)] = 1.0
    for j in range(7):
        for m in range(7):
            M[544 + 16 * j + m, WE + 649 + 13 * j + m] = 1.0
    for j in range(9):
        nb = min(4, 33 - 4 * j)
        for m in range(16):
            for b in range(nb):
                M[656 + 16 * j + m, WE + 555 + 4 * j + b] = float((m >> b) & 1)
    for i, (off, w) in enumerate([(597, 4), (601, 8), (625, 2), (627, 8),
                                  (635, 4), (639, 2), (641, 2)]):
        for m in range(w):
            M[800 + 16 * i + m, WE + off + m] = 1.0
    LC0 = _N + 7
    M[LC0 + 7, WE + 588] = 1.0 / 100
    M[LC0 + 8, WE + 589] = 1.0 / 1023
    for j in range(7):
        M[LC0 + 17 + j, WE + 590 + j] = 0.5
    return M


_M_NP = _mix_matrix()


def _build_luts(species_emb, abilities_emb, items_emb, actions_emb,
                ability_onehot, item_onehot, species_onehot,
                W_ab, W_it, W_enc):
    G = jnp.concatenate([W_enc, species_emb[:64], abilities_emb[:64],
                         items_emb[:64], actions_emb[:64], W_ab[:64],
                         W_it[:64]])
    La = jnp.asarray(_M_NP) @ G
    return La[:_N], La[_N + 7:]


def _encoder_block(e_ref, s_ref, colv_ref, andm_ref, clampm_ref,
                   sqrtm_ref, l_ref, lc_ref, bias_ref, scale_ref,
                   lnb_ref, o_ref):
    raw = e_ref[...]
    t = jnp.minimum(raw & andm_ref[...], clampm_ref[...])
    sq = ((raw >= 1).astype(jnp.int32) + (raw >= 4).astype(jnp.int32)
          + (raw >= 9).astype(jnp.int32) + (raw >= 16).astype(jnp.int32)
          + (raw >= 25).astype(jnp.int32) + (raw >= 36).astype(jnp.int32)
          + (raw >= 49).astype(jnp.int32))
    t = jnp.where(sqrtm_ref[...] == 1, sq, t)
    E = jnp.dot(t.astype(jnp.bfloat16), s_ref[...],
                preferred_element_type=jnp.float32)
    oh = (E == colv_ref[...]).astype(jnp.float32)
    acc = jnp.broadcast_to(bias_ref[...], (_B, _D))
    acc = acc + jnp.dot(raw.astype(jnp.float32), lc_ref[...],
                        preferred_element_type=jnp.float32)
    acc = acc + jnp.dot(oh, l_ref[...], preferred_element_type=jnp.float32)
    mu = jnp.mean(acc, axis=1, keepdims=True)
    d = acc - mu
    var = jnp.mean(d * d, axis=1, keepdims=True)
    o_ref[...] = d * jax.lax.rsqrt(var + 1e-6) * scale_ref[...] + lnb_ref[...]


def kernel(entity, species_emb, abilities_emb, items_emb, actions_emb,
           ability_onehot, item_onehot, species_onehot, W_ab, b_ab,
           W_it, b_it, W_enc, b_enc, ln_scale, ln_bias):
    L, Lc = _build_luts(species_emb, abilities_emb, items_emb, actions_emb,
                        ability_onehot, item_onehot, species_onehot,
                        W_ab, W_it, W_enc)
    bias = (b_ab + b_it + b_enc).reshape(1, _D)
    scale = ln_scale.reshape(1, _D)
    lnb = ln_bias.reshape(1, _D)
    S = jnp.asarray(_S_NP, jnp.bfloat16)
    colv = jnp.asarray(_COLV_NP).reshape(1, _N)
    andm = jnp.asarray(_ANDM_NP).reshape(1, _NF)
    clampm = jnp.asarray(_CLAMPM_NP).reshape(1, _NF)
    sqrtm = jnp.asarray(_SQRTM_NP).reshape(1, _NF)
    const = lambda i: (0, 0)
    return pl.pallas_call(
        _encoder_block,
        grid=(_BATCH // _B,),
        in_specs=[
            pl.BlockSpec((_B, _NF), lambda i: (i, 0)),
            pl.BlockSpec((_NF, _N), const),
            pl.BlockSpec((1, _N), const),
            pl.BlockSpec((1, _NF), const),
            pl.BlockSpec((1, _NF), const),
            pl.BlockSpec((1, _NF), const),
            pl.BlockSpec((_N, _D), const),
            pl.BlockSpec((_NF, _D), const),
            pl.BlockSpec((1, _D), const),
            pl.BlockSpec((1, _D), const),
            pl.BlockSpec((1, _D), const),
        ],
        out_specs=pl.BlockSpec((_B, _D), lambda i: (i, 0)),
        out_shape=jax.ShapeDtypeStruct((_BATCH, _D), jnp.float32),
    )(entity, S, colv, andm, clampm, sqrtm, L, Lc, bias, scale, lnb)


# speedup vs baseline: 2.4014x; 1.3658x over previous
import numpy as np
import jax
import jax.numpy as jnp
from jax.experimental import pallas as pl

_BATCH = 16384
_D = 256
_NF = 33
_B = 512
_N = 1024

_SRC64 = [0, 1, 2, 3, 4, 5, 6, 11]
_SRC16 = ([7, 8] + [17 + j for j in range(7)] + [24 + j for j in range(9)]
          + [9, 10, 12, 13, 14, 15, 16])


def _static_maps():
    src = np.full(_N, -1, np.int64)
    colv = np.full(_N, -1.0, np.float32)
    for i, s in enumerate(_SRC64):
        src[64 * i:64 * (i + 1)] = s
        colv[64 * i:64 * (i + 1)] = np.arange(64)
    for j, s in enumerate(_SRC16):
        b = 512 + 16 * j
        src[b:b + 16] = s
        colv[b:b + 16] = np.arange(16)
    S = np.zeros((_NF, _N), np.float32)
    valid = src >= 0
    S[src[valid], np.where(valid)[0]] = 1.0
    andm = np.full(_NF, 63, np.int32)
    andm[24:33] = 15
    clampm = np.full(_NF, 63, np.int32)
    for c, lim in [(9, 4), (10, 8), (12, 2), (13, 8), (14, 4), (15, 2),
                   (16, 2)]:
        clampm[c] = lim
    clampm[17:24] = 7
    sqrtm = np.zeros(_NF, np.int32)
    sqrtm[7] = sqrtm[8] = 1
    return S, colv, andm, clampm, sqrtm


_S_NP, _COLV_NP, _ANDM_NP, _CLAMPM_NP, _SQRTM_NP = _static_maps()

def _mix_matrix():
    WE, SP, AB, IT, ACT, WAB, WIT = 0, 734, 798, 862, 926, 990, 1054
    M = np.zeros((_N + _NF + 7, 1118), np.float32)
    for v in range(64):
        M[v, SP + v] = 1.0
        M[v, WE + v] = 1.0
        M[64 + v, AB + v] = 1.0
        M[64 + v, IT + v] = 1.0
        M[64 + v, WAB + v] = 1.0
        M[128 + v, WIT + v] = 1.0
        for k in range(4):
            M[192 + 64 * k + v, ACT + v] = 1.0
        if v < 16:
            M[448 + v, WE + 609 + v] = 1.0
    for s in range(16):
        M[512 + s, WE + 512 + min(s, 10)] = 1.0
        M[528 + s, WE + 523 + min(s, 31)] = 1.0
    for j in range(7):
        for m in range(7):
            M[544 + 16 * j + m, WE + 649 + 13 * j + m] = 1.0
    for j in range(9):
        nb = min(4, 33 - 4 * j)
        for m in range(16):
            for b in range(nb):
                M[656 + 16 * j + m, WE + 555 + 4 * j + b] = float((m >> b) & 1)
    for i, (off, w) in enumerate([(597, 4), (601, 8), (625, 2), (627, 8),
                                  (635, 4), (639, 2), (641, 2)]):
        for m in range(w):
            M[800 + 16 * i + m, WE + off + m] = 1.0
    LC0 = _N + 7
    M[LC0 + 7, WE + 588] = 1.0 / 100
    M[LC0 + 8, WE + 589] = 1.0 / 1023
    for j in range(7):
        M[LC0 + 17 + j, WE + 590 + j] = 0.5
    return M


_M_NP = _mix_matrix()


def _build_luts(species_emb, abilities_emb, items_emb, actions_emb,
                ability_onehot, item_onehot, species_onehot,
                W_ab, W_it, W_enc):
    G = jnp.concatenate([W_enc, species_emb[:64], abilities_emb[:64],
                         items_emb[:64], actions_emb[:64], W_ab[:64],
                         W_it[:64]])
    La = jnp.asarray(_M_NP) @ G
    return La[:_N], La[_N + 7:]


def _encoder_block(e_ref, s_ref, colv_ref, andm_ref, clampm_ref,
                   sqrtm_ref, l_ref, lc_ref, bias_ref, scale_ref,
                   lnb_ref, o_ref):
    raw = e_ref[...]
    t = jnp.minimum(raw & andm_ref[...], clampm_ref[...])
    sq = ((raw >= 1).astype(jnp.int32) + (raw >= 4).astype(jnp.int32)
          + (raw >= 9).astype(jnp.int32) + (raw >= 16).astype(jnp.int32)
          + (raw >= 25).astype(jnp.int32) + (raw >= 36).astype(jnp.int32)
          + (raw >= 49).astype(jnp.int32))
    t = jnp.where(sqrtm_ref[...] == 1, sq, t)
    E = jnp.dot(t.astype(jnp.bfloat16), s_ref[...],
                preferred_element_type=jnp.float32)
    oh = (E == colv_ref[...]).astype(jnp.float32)
    acc = jnp.broadcast_to(bias_ref[...], (_B, _D))
    acc = acc + jnp.dot(raw.astype(jnp.float32), lc_ref[...],
                        preferred_element_type=jnp.float32)
    acc = acc + jnp.dot(oh, l_ref[...], preferred_element_type=jnp.float32)
    mu = jnp.mean(acc, axis=1, keepdims=True)
    d = acc - mu
    var = jnp.mean(d * d, axis=1, keepdims=True)
    o_ref[...] = d * jax.lax.rsqrt(var + 1e-6) * scale_ref[...] + lnb_ref[...]


def kernel(entity, species_emb, abilities_emb, items_emb, actions_emb,
           ability_onehot, item_onehot, species_onehot, W_ab, b_ab,
           W_it, b_it, W_enc, b_enc, ln_scale, ln_bias):
    L, Lc = _build_luts(species_emb, abilities_emb, items_emb, actions_emb,
                        ability_onehot, item_onehot, species_onehot,
                        W_ab, W_it, W_enc)
    bias = (b_ab + b_it + b_enc).reshape(1, _D)
    scale = ln_scale.reshape(1, _D)
    lnb = ln_bias.reshape(1, _D)
    S = jnp.asarray(_S_NP, jnp.bfloat16)
    colv = jnp.asarray(_COLV_NP).reshape(1, _N)
    andm = jnp.asarray(_ANDM_NP).reshape(1, _NF)
    clampm = jnp.asarray(_CLAMPM_NP).reshape(1, _NF)
    sqrtm = jnp.asarray(_SQRTM_NP).reshape(1, _NF)
    const = lambda i: (0, 0)
    return pl.pallas_call(
        _encoder_block,
        grid=(_BATCH // _B,),
        in_specs=[
            pl.BlockSpec((_B, _NF), lambda i: (i, 0)),
            pl.BlockSpec((_NF, _N), const),
            pl.BlockSpec((1, _N), const),
            pl.BlockSpec((1, _NF), const),
            pl.BlockSpec((1, _NF), const),
            pl.BlockSpec((1, _NF), const),
            pl.BlockSpec((_N, _D), const),
            pl.BlockSpec((_NF, _D), const),
            pl.BlockSpec((1, _D), const),
            pl.BlockSpec((1, _D), const),
            pl.BlockSpec((1, _D), const),
        ],
        out_specs=pl.BlockSpec((_B, _D), lambda i: (i, 0)),
        out_shape=jax.ShapeDtypeStruct((_BATCH, _D), jnp.float32),
    )(entity, S, colv, andm, clampm, sqrtm, L, Lc, bias, scale, lnb)


# speedup vs baseline: 2.7655x; 1.1516x over previous
import numpy as np
import jax
import jax.numpy as jnp
from jax.experimental import pallas as pl

_BATCH = 16384
_D = 256
_NF = 33
_B = 1024
_N = 1024

_SRC64 = [0, 1, 2, 3, 4, 5, 6, 11]
_SRC16 = ([7, 8] + [17 + j for j in range(7)] + [24 + j for j in range(9)]
          + [9, 10, 12, 13, 14, 15, 16])


def _static_maps():
    src = np.full(_N, -1, np.int64)
    colv = np.full(_N, -1.0, np.float32)
    for i, s in enumerate(_SRC64):
        src[64 * i:64 * (i + 1)] = s
        colv[64 * i:64 * (i + 1)] = np.arange(64)
    for j, s in enumerate(_SRC16):
        b = 512 + 16 * j
        src[b:b + 16] = s
        colv[b:b + 16] = np.arange(16)
    S = np.zeros((_NF, _N), np.float32)
    valid = src >= 0
    S[src[valid], np.where(valid)[0]] = 1.0
    andm = np.full(_NF, 63, np.int32)
    andm[24:33] = 15
    clampm = np.full(_NF, 63, np.int32)
    for c, lim in [(9, 4), (10, 8), (12, 2), (13, 8), (14, 4), (15, 2),
                   (16, 2)]:
        clampm[c] = lim
    clampm[17:24] = 7
    sqrtm = np.zeros(_NF, np.int32)
    sqrtm[7] = sqrtm[8] = 1
    return S, colv, andm, clampm, sqrtm


_S_NP, _COLV_NP, _ANDM_NP, _CLAMPM_NP, _SQRTM_NP = _static_maps()

def _mix_matrix():
    WE, SP, AB, IT, ACT, WAB, WIT = 0, 734, 798, 862, 926, 990, 1054
    M = np.zeros((_N + _NF + 7, 1118), np.float32)
    for v in range(64):
        M[v, SP + v] = 1.0
        M[v, WE + v] = 1.0
        M[64 + v, AB + v] = 1.0
        M[64 + v, IT + v] = 1.0
        M[64 + v, WAB + v] = 1.0
        M[128 + v, WIT + v] = 1.0
        for k in range(4):
            M[192 + 64 * k + v, ACT + v] = 1.0
        if v < 16:
            M[448 + v, WE + 609 + v] = 1.0
    for s in range(16):
        M[512 + s, WE + 512 + min(s, 10)] = 1.0
        M[528 + s, WE + 523 + min(s, 31)] = 1.0
    for j in range(7):
        for m in range(7):
            M[544 + 16 * j + m, WE + 649 + 13 * j + m] = 1.0
    for j in range(9):
        nb = min(4, 33 - 4 * j)
        for m in range(16):
            for b in range(nb):
                M[656 + 16 * j + m, WE + 555 + 4 * j + b] = float((m >> b) & 1)
    for i, (off, w) in enumerate([(597, 4), (601, 8), (625, 2), (627, 8),
                                  (635, 4), (639, 2), (641, 2)]):
        for m in range(w):
            M[800 + 16 * i + m, WE + off + m] = 1.0
    LC0 = _N + 7
    M[LC0 + 7, WE + 588] = 1.0 / 100
    M[LC0 + 8, WE + 589] = 1.0 / 1023
    for j in range(7):
        M[LC0 + 17 + j, WE + 590 + j] = 0.5
    return M


_M_NP = _mix_matrix()


def _build_luts(species_emb, abilities_emb, items_emb, actions_emb,
                ability_onehot, item_onehot, species_onehot,
                W_ab, W_it, W_enc):
    G = jnp.concatenate([W_enc, species_emb[:64], abilities_emb[:64],
                         items_emb[:64], actions_emb[:64], W_ab[:64],
                         W_it[:64]])
    La = jnp.asarray(_M_NP) @ G
    return La[:_N], La[_N + 7:]


def _encoder_block(e_ref, s_ref, colv_ref, andm_ref, clampm_ref,
                   sqrtm_ref, l_ref, lc_ref, bias_ref, scale_ref,
                   lnb_ref, o_ref):
    raw = e_ref[...]
    t = jnp.minimum(raw & andm_ref[...], clampm_ref[...])
    sq = ((raw >= 1).astype(jnp.int32) + (raw >= 4).astype(jnp.int32)
          + (raw >= 9).astype(jnp.int32) + (raw >= 16).astype(jnp.int32)
          + (raw >= 25).astype(jnp.int32) + (raw >= 36).astype(jnp.int32)
          + (raw >= 49).astype(jnp.int32))
    t = jnp.where(sqrtm_ref[...] == 1, sq, t)
    E = jnp.dot(t.astype(jnp.bfloat16), s_ref[...],
                preferred_element_type=jnp.float32)
    oh = (E == colv_ref[...]).astype(jnp.float32)
    acc = jnp.broadcast_to(bias_ref[...], (_B, _D))
    acc = acc + jnp.dot(raw.astype(jnp.float32), lc_ref[...],
                        preferred_element_type=jnp.float32)
    acc = acc + jnp.dot(oh, l_ref[...], preferred_element_type=jnp.float32)
    mu = jnp.mean(acc, axis=1, keepdims=True)
    d = acc - mu
    var = jnp.mean(d * d, axis=1, keepdims=True)
    o_ref[...] = d * jax.lax.rsqrt(var + 1e-6) * scale_ref[...] + lnb_ref[...]


def kernel(entity, species_emb, abilities_emb, items_emb, actions_emb,
           ability_onehot, item_onehot, species_onehot, W_ab, b_ab,
           W_it, b_it, W_enc, b_enc, ln_scale, ln_bias):
    L, Lc = _build_luts(species_emb, abilities_emb, items_emb, actions_emb,
                        ability_onehot, item_onehot, species_onehot,
                        W_ab, W_it, W_enc)
    bias = (b_ab + b_it + b_enc).reshape(1, _D)
    scale = ln_scale.reshape(1, _D)
    lnb = ln_bias.reshape(1, _D)
    S = jnp.asarray(_S_NP, jnp.bfloat16)
    colv = jnp.asarray(_COLV_NP).reshape(1, _N)
    andm = jnp.asarray(_ANDM_NP).reshape(1, _NF)
    clampm = jnp.asarray(_CLAMPM_NP).reshape(1, _NF)
    sqrtm = jnp.asarray(_SQRTM_NP).reshape(1, _NF)
    const = lambda i: (0, 0)
    return pl.pallas_call(
        _encoder_block,
        grid=(_BATCH // _B,),
        in_specs=[
            pl.BlockSpec((_B, _NF), lambda i: (i, 0)),
            pl.BlockSpec((_NF, _N), const),
            pl.BlockSpec((1, _N), const),
            pl.BlockSpec((1, _NF), const),
            pl.BlockSpec((1, _NF), const),
            pl.BlockSpec((1, _NF), const),
            pl.BlockSpec((_N, _D), const),
            pl.BlockSpec((_NF, _D), const),
            pl.BlockSpec((1, _D), const),
            pl.BlockSpec((1, _D), const),
            pl.BlockSpec((1, _D), const),
        ],
        out_specs=pl.BlockSpec((_B, _D), lambda i: (i, 0)),
        out_shape=jax.ShapeDtypeStruct((_BATCH, _D), jnp.float32),
    )(entity, S, colv, andm, clampm, sqrtm, L, Lc, bias, scale, lnb)


# speedup vs baseline: 2.8053x; 1.0144x over previous
import numpy as np
import jax
import jax.numpy as jnp
from jax.experimental import pallas as pl

_BATCH = 16384
_D = 256
_NF = 33
_B = 2048
_N = 1024

_SRC64 = [0, 1, 2, 3, 4, 5, 6, 11]
_SRC16 = ([7, 8] + [17 + j for j in range(7)] + [24 + j for j in range(9)]
          + [9, 10, 12, 13, 14, 15, 16])


def _static_maps():
    src = np.full(_N, -1, np.int64)
    colv = np.full(_N, -1.0, np.float32)
    for i, s in enumerate(_SRC64):
        src[64 * i:64 * (i + 1)] = s
        colv[64 * i:64 * (i + 1)] = np.arange(64)
    for j, s in enumerate(_SRC16):
        b = 512 + 16 * j
        src[b:b + 16] = s
        colv[b:b + 16] = np.arange(16)
    S = np.zeros((_NF, _N), np.float32)
    valid = src >= 0
    S[src[valid], np.where(valid)[0]] = 1.0
    andm = np.full(_NF, 63, np.int32)
    andm[24:33] = 15
    clampm = np.full(_NF, 63, np.int32)
    for c, lim in [(9, 4), (10, 8), (12, 2), (13, 8), (14, 4), (15, 2),
                   (16, 2)]:
        clampm[c] = lim
    clampm[17:24] = 7
    sqrtm = np.zeros(_NF, np.int32)
    sqrtm[7] = sqrtm[8] = 1
    return S, colv, andm, clampm, sqrtm


_S_NP, _COLV_NP, _ANDM_NP, _CLAMPM_NP, _SQRTM_NP = _static_maps()

def _mix_matrix():
    WE, SP, AB, IT, ACT, WAB, WIT = 0, 734, 798, 862, 926, 990, 1054
    M = np.zeros((_N + _NF + 7, 1118), np.float32)
    for v in range(64):
        M[v, SP + v] = 1.0
        M[v, WE + v] = 1.0
        M[64 + v, AB + v] = 1.0
        M[64 + v, IT + v] = 1.0
        M[64 + v, WAB + v] = 1.0
        M[128 + v, WIT + v] = 1.0
        for k in range(4):
            M[192 + 64 * k + v, ACT + v] = 1.0
        if v < 16:
            M[448 + v, WE + 609 + v] = 1.0
    for s in range(16):
        M[512 + s, WE + 512 + min(s, 10)] = 1.0
        M[528 + s, WE + 523 + min(s, 31)] = 1.0
    for j in range(7):
        for m in range(7):
            M[544 + 16 * j + m, WE + 649 + 13 * j + m] = 1.0
    for j in range(9):
        nb = min(4, 33 - 4 * j)
        for m in range(16):
            for b in range(nb):
                M[656 + 16 * j + m, WE + 555 + 4 * j + b] = float((m >> b) & 1)
    for i, (off, w) in enumerate([(597, 4), (601, 8), (625, 2), (627, 8),
                                  (635, 4), (639, 2), (641, 2)]):
        for m in range(w):
            M[800 + 16 * i + m, WE + off + m] = 1.0
    LC0 = _N + 7
    M[LC0 + 7, WE + 588] = 1.0 / 100
    M[LC0 + 8, WE + 589] = 1.0 / 1023
    for j in range(7):
        M[LC0 + 17 + j, WE + 590 + j] = 0.5
    return M


_M_NP = _mix_matrix()


def _build_luts(species_emb, abilities_emb, items_emb, actions_emb,
                ability_onehot, item_onehot, species_onehot,
                W_ab, W_it, W_enc):
    G = jnp.concatenate([W_enc, species_emb[:64], abilities_emb[:64],
                         items_emb[:64], actions_emb[:64], W_ab[:64],
                         W_it[:64]])
    La = jnp.asarray(_M_NP) @ G
    return La[:_N], La[_N + 7:]


def _encoder_block(e_ref, s_ref, colv_ref, andm_ref, clampm_ref,
                   sqrtm_ref, l_ref, lc_ref, bias_ref, scale_ref,
                   lnb_ref, o_ref):
    raw = e_ref[...]
    t = jnp.minimum(raw & andm_ref[...], clampm_ref[...])
    sq = ((raw >= 1).astype(jnp.int32) + (raw >= 4).astype(jnp.int32)
          + (raw >= 9).astype(jnp.int32) + (raw >= 16).astype(jnp.int32)
          + (raw >= 25).astype(jnp.int32) + (raw >= 36).astype(jnp.int32)
          + (raw >= 49).astype(jnp.int32))
    t = jnp.where(sqrtm_ref[...] == 1, sq, t)
    E = jnp.dot(t.astype(jnp.bfloat16), s_ref[...],
                preferred_element_type=jnp.float32)
    oh = (E == colv_ref[...]).astype(jnp.float32)
    acc = jnp.broadcast_to(bias_ref[...], (_B, _D))
    acc = acc + jnp.dot(raw.astype(jnp.float32), lc_ref[...],
                        preferred_element_type=jnp.float32)
    acc = acc + jnp.dot(oh, l_ref[...], preferred_element_type=jnp.float32)
    mu = jnp.mean(acc, axis=1, keepdims=True)
    d = acc - mu
    var = jnp.mean(d * d, axis=1, keepdims=True)
    o_ref[...] = d * jax.lax.rsqrt(var + 1e-6) * scale_ref[...] + lnb_ref[...]


def kernel(entity, species_emb, abilities_emb, items_emb, actions_emb,
           ability_onehot, item_onehot, species_onehot, W_ab, b_ab,
           W_it, b_it, W_enc, b_enc, ln_scale, ln_bias):
    L, Lc = _build_luts(species_emb, abilities_emb, items_emb, actions_emb,
                        ability_onehot, item_onehot, species_onehot,
                        W_ab, W_it, W_enc)
    bias = (b_ab + b_it + b_enc).reshape(1, _D)
    scale = ln_scale.reshape(1, _D)
    lnb = ln_bias.reshape(1, _D)
    S = jnp.asarray(_S_NP, jnp.bfloat16)
    colv = jnp.asarray(_COLV_NP).reshape(1, _N)
    andm = jnp.asarray(_ANDM_NP).reshape(1, _NF)
    clampm = jnp.asarray(_CLAMPM_NP).reshape(1, _NF)
    sqrtm = jnp.asarray(_SQRTM_NP).reshape(1, _NF)
    const = lambda i: (0, 0)
    return pl.pallas_call(
        _encoder_block,
        grid=(_BATCH // _B,),
        in_specs=[
            pl.BlockSpec((_B, _NF), lambda i: (i, 0)),
            pl.BlockSpec((_NF, _N), const),
            pl.BlockSpec((1, _N), const),
            pl.BlockSpec((1, _NF), const),
            pl.BlockSpec((1, _NF), const),
            pl.BlockSpec((1, _NF), const),
            pl.BlockSpec((_N, _D), const),
            pl.BlockSpec((_NF, _D), const),
            pl.BlockSpec((1, _D), const),
            pl.BlockSpec((1, _D), const),
            pl.BlockSpec((1, _D), const),
        ],
        out_specs=pl.BlockSpec((_B, _D), lambda i: (i, 0)),
        out_shape=jax.ShapeDtypeStruct((_BATCH, _D), jnp.float32),
    )(entity, S, colv, andm, clampm, sqrtm, L, Lc, bias, scale, lnb)
